# baseline (speedup 1.0000x reference)
import dataclasses

import jax
import jax.numpy as jnp
from jax import lax
from jax.experimental import pallas as pl
from jax.experimental import pallas as _pl
from jax.experimental.pallas import tpu as pltpu
from jax.experimental.pallas import tpu_sc as plsc

GAMMA = 12.0
EMB_RANGE = 0.21875
PI = 3.141592653589793
PHASE_SCALE = PI / EMB_RANGE

B = 4096
D = 64
ED = 128
NREL = 1000
NREL_PAD = 1024
NC, NS = 2, 16
NW = NC * NS
BPW = B // NW


def _phase_table_body(r_ref, o_ref):
    ph = r_ref[...] * PHASE_SCALE
    o_ref[pl.ds(0, NREL), pl.ds(0, D)] = jnp.cos(ph)
    o_ref[pl.ds(0, NREL), pl.ds(D, D)] = jnp.sin(ph)


def _extract_col(smp_v, col, out_v):
    row_iota = lax.iota(jnp.int32, 16)
    col_vec = jnp.full(16, col, jnp.int32)
    for k in range(BPW // 16):
        rows = row_iota + (16 * k)
        out_v[pl.ds(16 * k, 16)] = plsc.load_gather(smp_v, [rows, col_vec])


def _gather_ht_body(sample_hbm, ent_hbm, hout_hbm, tout_hbm,
                    smp_v, hidx_v, tidx_v, hbuf, tbuf, s1, s2, s3, s4):
    wid = lax.axis_index("s") * NC + lax.axis_index("c")
    base = wid * BPW
    pltpu.sync_copy(sample_hbm.at[pl.ds(base, BPW)], smp_v)
    _extract_col(smp_v, 0, hidx_v)
    _extract_col(smp_v, 2, tidx_v)
    gh = pltpu.async_copy(ent_hbm.at[hidx_v], hbuf, s1)
    gt = pltpu.async_copy(ent_hbm.at[tidx_v], tbuf, s2)
    gh.wait()
    wh = pltpu.async_copy(hbuf, hout_hbm.at[pl.ds(base, BPW)], s3)
    gt.wait()
    wt = pltpu.async_copy(tbuf, tout_hbm.at[pl.ds(base, BPW)], s4)
    wh.wait()
    wt.wait()


def _gather_cs_body(sample_hbm, cs_hbm, cout_hbm,
                    smp_v, ridx_v, rbuf, s1, s2):
    wid = lax.axis_index("s") * NC + lax.axis_index("c")
    base = wid * BPW
    pltpu.sync_copy(sample_hbm.at[pl.ds(base, BPW)], smp_v)
    _extract_col(smp_v, 1, ridx_v)
    gr = pltpu.async_copy(cs_hbm.at[ridx_v], rbuf, s1)
    gr.wait()
    pltpu.async_copy(rbuf, cout_hbm.at[pl.ds(base, BPW)], s2).wait()


def _score_body(h_ref, t_ref, c_ref, o_ref):
    re_h = h_ref[:, :D]
    im_h = h_ref[:, D:]
    re_t = t_ref[:, :D]
    im_t = t_ref[:, D:]
    re_r = c_ref[:, :D]
    im_r = c_ref[:, D:]
    re_s = re_h * re_r - im_h * im_r - re_t
    im_s = re_h * im_r + im_h * re_r - im_t
    mag = jnp.sqrt(re_s * re_s + im_s * im_s)
    o_ref[...] = GAMMA - jnp.sum(mag, axis=1, keepdims=True)


def kernel(sample, entity_embedding, relation_embedding):
    sample = sample.astype(jnp.int32)
    f32 = jnp.float32
    mesh = plsc.VectorSubcoreMesh(core_axis_name="c", subcore_axis_name="s")
    cp = pltpu.CompilerParams()
    if "needs_layout_passes" in pltpu.CompilerParams.__dataclass_fields__:
        cp = dataclasses.replace(cp, needs_layout_passes=False)

    gather_ht = pl.kernel(
        _gather_ht_body,
        compiler_params=cp,
        out_type=(jax.ShapeDtypeStruct((B, ED), f32),
                  jax.ShapeDtypeStruct((B, ED), f32)),
        mesh=mesh,
        scratch_types=[
            pltpu.VMEM((BPW, 3), jnp.int32),
            pltpu.VMEM((BPW,), jnp.int32),
            pltpu.VMEM((BPW,), jnp.int32),
            pltpu.VMEM((BPW, ED), f32),
            pltpu.VMEM((BPW, ED), f32),
            pltpu.SemaphoreType.DMA,
            pltpu.SemaphoreType.DMA,
            pltpu.SemaphoreType.DMA,
            pltpu.SemaphoreType.DMA,
        ],
    )
    hrows, trows = gather_ht(sample, entity_embedding)

    cossin = pl.pallas_call(
        _phase_table_body,
        out_shape=jax.ShapeDtypeStruct((NREL_PAD, ED), f32),
    )(relation_embedding)

    gather_cs = pl.kernel(
        _gather_cs_body,
        compiler_params=cp,
        out_type=jax.ShapeDtypeStruct((B, ED), f32),
        mesh=mesh,
        scratch_types=[
            pltpu.VMEM((BPW, 3), jnp.int32),
            pltpu.VMEM((BPW,), jnp.int32),
            pltpu.VMEM((BPW, ED), f32),
            pltpu.SemaphoreType.DMA,
            pltpu.SemaphoreType.DMA,
        ],
    )
    csrows = gather_cs(sample, cossin)

    nblk = 4
    blk = B // nblk
    score = pl.pallas_call(
        _score_body,
        out_shape=jax.ShapeDtypeStruct((B, 1), f32),
        grid=(nblk,),
        in_specs=[
            pl.BlockSpec((blk, ED), lambda i: (i, 0)),
            pl.BlockSpec((blk, ED), lambda i: (i, 0)),
            pl.BlockSpec((blk, ED), lambda i: (i, 0)),
        ],
        out_specs=pl.BlockSpec((blk, 1), lambda i: (i, 0)),
    )(hrows, trows, csrows)
    return score


# speedup vs baseline: 1.3091x; 1.3091x over previous
import jax
import jax.numpy as jnp
from jax import lax
from jax.experimental import pallas as pl
from jax.experimental.pallas import tpu as pltpu
from jax.experimental.pallas import tpu_sc as plsc

GAMMA = 12.0
EMB_RANGE = 0.21875
PI = 3.141592653589793
PHASE_SCALE = PI / EMB_RANGE

B = 4096
D = 64
ED = 128
NREL = 1000
NREL_PAD = 1024
NC, NS = 2, 16
NW = NC * NS
BPW = B // NW


def _phase_table_body(r_ref, o_ref):
    ph = r_ref[...] * PHASE_SCALE
    o_ref[pl.ds(0, NREL), pl.ds(0, D)] = jnp.cos(ph)
    o_ref[pl.ds(0, NREL), pl.ds(D, D)] = jnp.sin(ph)


def _gather_body(ent_hbm, cs_hbm, hidx_hbm, tidx_hbm, ridx_hbm,
                 hout_hbm, tout_hbm, cout_hbm,
                 hidx_v, tidx_v, ridx_v, hbuf, tbuf, rbuf,
                 s1, s2, s3, s4, s5, s6):
    wid = lax.axis_index("s") * NC + lax.axis_index("c")
    base = wid * BPW
    pltpu.sync_copy(hidx_hbm.at[pl.ds(base, BPW)], hidx_v)
    pltpu.sync_copy(tidx_hbm.at[pl.ds(base, BPW)], tidx_v)
    pltpu.sync_copy(ridx_hbm.at[pl.ds(base, BPW)], ridx_v)
    gh = pltpu.async_copy(ent_hbm.at[hidx_v], hbuf, s1)
    gt = pltpu.async_copy(ent_hbm.at[tidx_v], tbuf, s2)
    gr = pltpu.async_copy(cs_hbm.at[ridx_v], rbuf, s3)
    gh.wait()
    wh = pltpu.async_copy(hbuf, hout_hbm.at[pl.ds(base, BPW)], s4)
    gt.wait()
    wt = pltpu.async_copy(tbuf, tout_hbm.at[pl.ds(base, BPW)], s5)
    gr.wait()
    wr = pltpu.async_copy(rbuf, cout_hbm.at[pl.ds(base, BPW)], s6)
    wh.wait()
    wt.wait()
    wr.wait()


def _score_body(h_ref, t_ref, c_ref, o_ref):
    re_h = h_ref[:, :D]
    im_h = h_ref[:, D:]
    re_t = t_ref[:, :D]
    im_t = t_ref[:, D:]
    re_r = c_ref[:, :D]
    im_r = c_ref[:, D:]
    re_s = re_h * re_r - im_h * im_r - re_t
    im_s = re_h * im_r + im_h * re_r - im_t
    mag = jnp.sqrt(re_s * re_s + im_s * im_s)
    s = GAMMA - jnp.sum(mag, axis=1)
    o_ref[...] = s.reshape(o_ref.shape)


def kernel(sample, entity_embedding, relation_embedding):
    sample = sample.astype(jnp.int32)
    hidx = sample[:, 0]
    tidx = sample[:, 2]
    ridx = sample[:, 1]
    f32 = jnp.float32

    cossin = pl.pallas_call(
        _phase_table_body,
        out_shape=jax.ShapeDtypeStruct((NREL_PAD, ED), f32),
    )(relation_embedding)

    mesh = plsc.VectorSubcoreMesh(core_axis_name="c", subcore_axis_name="s")
    gather = pl.kernel(
        _gather_body,
        out_type=(jax.ShapeDtypeStruct((B, ED), f32),
                  jax.ShapeDtypeStruct((B, ED), f32),
                  jax.ShapeDtypeStruct((B, ED), f32)),
        mesh=mesh,
        scratch_types=[
            pltpu.VMEM((BPW,), jnp.int32),
            pltpu.VMEM((BPW,), jnp.int32),
            pltpu.VMEM((BPW,), jnp.int32),
            pltpu.VMEM((BPW, ED), f32),
            pltpu.VMEM((BPW, ED), f32),
            pltpu.VMEM((BPW, ED), f32),
            pltpu.SemaphoreType.DMA,
            pltpu.SemaphoreType.DMA,
            pltpu.SemaphoreType.DMA,
            pltpu.SemaphoreType.DMA,
            pltpu.SemaphoreType.DMA,
            pltpu.SemaphoreType.DMA,
        ],
    )
    hrows, trows, csrows = gather(entity_embedding, cossin, hidx, tidx, ridx)

    nblk = 4
    blk = B // nblk
    score = pl.pallas_call(
        _score_body,
        out_shape=jax.ShapeDtypeStruct((B // ED, ED), f32),
        grid=(nblk,),
        in_specs=[
            pl.BlockSpec((blk, ED), lambda i: (i, 0)),
            pl.BlockSpec((blk, ED), lambda i: (i, 0)),
            pl.BlockSpec((blk, ED), lambda i: (i, 0)),
        ],
        out_specs=pl.BlockSpec((blk // ED, ED), lambda i: (i, 0)),
    )(hrows, trows, csrows)
    return score.reshape(B, 1)
